# Initial kernel scaffold; baseline (speedup 1.0000x reference)
import jax
import jax.numpy as jnp
from jax.experimental import pallas as pl

N_BEH = 8192
N_OBJ = 256
N_PROP = 16
STEP_DIST = 0.05
BLK = 256


def _body(xt_ref, p0_ref, p1_ref, md_ref, dt_ref, xtype_ref, ytype_ref,
          om_ref, w_ref, out_ref):
    xt = xt_ref[...]
    p0 = p0_ref[...]
    p1 = p1_ref[...]
    md = md_ref[...]
    dt = dt_ref[...]
    xtyp = xtype_ref[...]
    ytyp = ytype_ref[...]
    om = om_ref[...]
    w = w_ref[...]

    prop_iota = jax.lax.broadcasted_iota(jnp.int32, (BLK, N_PROP), 1)
    oh0 = (prop_iota == p0[:, None]).astype(jnp.float32)
    oh1 = (prop_iota == p1[:, None]).astype(jnp.float32)
    c0 = jnp.dot(oh0, xt, preferred_element_type=jnp.float32)
    c1 = jnp.dot(oh1, xt, preferred_element_type=jnp.float32)

    rad = md * (jnp.pi / 180.0)
    mvx = STEP_DIST * jnp.cos(rad)
    mvy = STEP_DIST * jnp.sin(rad)
    p1mx = c0[:, 0] + mvx
    p1my = c1[:, 0] + mvy

    dx = c0 - p1mx[:, None]
    dy = c1 - p1my[:, None]

    distx = jnp.round(jnp.abs(dx) / 0.05) * 0.05
    disty = jnp.round(jnp.abs(dy) / 0.05) * 0.05
    ang = jnp.arctan2(dy, dx) * (180.0 / jnp.pi)
    dirs = jnp.round(ang / 45.0) * 45.0

    mask = ((dirs == dt[:, None])
            & (distx == xtyp[:, None])
            & (disty == ytyp[:, None])
            & (om > 0.0))
    hit = jnp.sum(mask.astype(jnp.float32), axis=1) > 0.0
    out_ref[...] = hit.astype(jnp.float32) * w


def kernel(x, p, move_directions, dir_types, x_types, y_types, o_mask,
           beh_weights):
    xt = x[0].T
    obj_idx = jnp.arange(N_OBJ)[None, :]
    omf = jnp.where(o_mask & (obj_idx > 0), 1.0, 0.0).astype(jnp.float32)
    p0 = p[:, 0].astype(jnp.int32)
    p1 = p[:, 1].astype(jnp.int32)

    grid = (N_BEH // BLK,)
    beh_spec = pl.BlockSpec((BLK,), lambda i: (i,))
    return pl.pallas_call(
        _body,
        grid=grid,
        in_specs=[
            pl.BlockSpec((N_PROP, N_OBJ), lambda i: (0, 0)),
            beh_spec, beh_spec, beh_spec, beh_spec, beh_spec, beh_spec,
            pl.BlockSpec((BLK, N_OBJ), lambda i: (i, 0)),
            beh_spec,
        ],
        out_specs=beh_spec,
        out_shape=jax.ShapeDtypeStruct((N_BEH,), jnp.float32),
    )(xt, p0, p1, move_directions, dir_types, x_types, y_types, omf,
      beh_weights)


# speedup vs baseline: 2.2568x; 2.2568x over previous
import jax
import jax.numpy as jnp
from jax.experimental import pallas as pl

N_BEH = 8192
N_OBJ = 256
N_PROP = 16
STEP_DIST = 0.05
BLK = 256


def _body(xt_ref, p0_ref, p1_ref, md_ref, dt_ref, xtype_ref, ytype_ref,
          om_ref, w_ref, out_ref):
    xt = xt_ref[...]
    p0 = p0_ref[...]
    p1 = p1_ref[...]
    md = md_ref[...]
    dt = dt_ref[...]
    xtyp = xtype_ref[...]
    ytyp = ytype_ref[...]
    om = om_ref[...]
    w = w_ref[...]

    prop_iota = jax.lax.broadcasted_iota(jnp.int32, (BLK, N_PROP), 1)
    oh0 = (prop_iota == p0[:, None]).astype(jnp.float32)
    oh1 = (prop_iota == p1[:, None]).astype(jnp.float32)
    c0 = jnp.dot(oh0, xt, preferred_element_type=jnp.float32,
                 precision=jax.lax.Precision.HIGHEST)
    c1 = jnp.dot(oh1, xt, preferred_element_type=jnp.float32,
                 precision=jax.lax.Precision.HIGHEST)

    rad = md * (jnp.pi / 180.0)
    mvx = STEP_DIST * jnp.cos(rad)
    mvy = STEP_DIST * jnp.sin(rad)
    p1mx = c0[:, 0] + mvx
    p1my = c1[:, 0] + mvy

    dx = c0 - p1mx[:, None]
    dy = c1 - p1my[:, None]

    distx = jnp.round(jnp.abs(dx) / 0.05) * 0.05
    disty = jnp.round(jnp.abs(dy) / 0.05) * 0.05
    ang = jnp.arctan2(dy, dx) * (180.0 / jnp.pi)
    dirs = jnp.round(ang / 45.0) * 45.0

    mask = ((dirs == dt[:, None])
            & (distx == xtyp[:, None])
            & (disty == ytyp[:, None])
            & (om > 0.0))
    hit = jnp.sum(mask.astype(jnp.float32), axis=1) > 0.0
    out_ref[...] = hit.astype(jnp.float32) * w


def kernel(x, p, move_directions, dir_types, x_types, y_types, o_mask,
           beh_weights):
    xt = x[0].T
    obj_idx = jnp.arange(N_OBJ)[None, :]
    omf = jnp.where(o_mask & (obj_idx > 0), 1.0, 0.0).astype(jnp.float32)
    p0 = p[:, 0].astype(jnp.int32)
    p1 = p[:, 1].astype(jnp.int32)

    grid = (N_BEH // BLK,)
    beh_spec = pl.BlockSpec((BLK,), lambda i: (i,))
    return pl.pallas_call(
        _body,
        grid=grid,
        in_specs=[
            pl.BlockSpec((N_PROP, N_OBJ), lambda i: (0, 0)),
            beh_spec, beh_spec, beh_spec, beh_spec, beh_spec, beh_spec,
            pl.BlockSpec((BLK, N_OBJ), lambda i: (i, 0)),
            beh_spec,
        ],
        out_specs=beh_spec,
        out_shape=jax.ShapeDtypeStruct((N_BEH,), jnp.float32),
    )(xt, p0, p1, move_directions, dir_types, x_types, y_types, omf,
      beh_weights)
